# Initial kernel scaffold; baseline (speedup 1.0000x reference)
import jax
import jax.numpy as jnp
from jax.experimental import pallas as pl


def kernel(query, reference_points, value, value_spatial_shapes, Wv, bv, Ws, bso, Wa, ba, Wo, bo):
    raise NotImplementedError("write your pallas kernel here")


# speedup vs baseline: 879.0412x; 879.0412x over previous
import functools
import numpy as np
import jax
import jax.numpy as jnp
from jax import lax
from jax.experimental import pallas as pl
from jax.experimental.pallas import tpu as pltpu
from jax.experimental.pallas import tpu_sc as plsc

B = 4
LQ = 900
E = 256
H = 8
D = 32
SUM_P = 16
SPATIAL = ((128, 128), (64, 64), (32, 32), (16, 16))
LV = sum(h * w for h, w in SPATIAL)
NQ = B * LQ
NR = NQ * H
KPR = 4 * SUM_P
NK = NR * KPR
TROWS = B * LV * H

NWORK = 32
RPW = NR // NWORK
GSZ = 20
NGRP = RPW // GSZ
KG = GSZ * KPR
NCH = KG // 128

_lev_of_p = np.array([p // 4 for p in range(SUM_P)])
_W_of_lev = np.array([w for (h, w) in SPATIAL], dtype=np.float64)
_H_of_lev = np.array([h for (h, w) in SPATIAL], dtype=np.float64)
_base_of_lev = np.cumsum([0] + [h * w for (h, w) in SPATIAL])[:4]

_dim256 = np.zeros(256, np.float32)
for _j in range(256):
    _p = (_j % 32) // 2
    _l = _lev_of_p[_p]
    _dim256[_j] = _W_of_lev[_l] if _j % 2 == 0 else _H_of_lev[_l]
_xybit256 = np.array([_j % 2 for _j in range(256)], np.float32)

_cx512 = np.zeros(512, np.float32); _cy512 = np.zeros(512, np.float32)
_Wl512 = np.zeros(512, np.float32); _Hl512 = np.zeros(512, np.float32)
_base512 = np.zeros(512, np.float32); _h512 = np.zeros(512, np.float32)
for _j in range(512):
    _h = _j // 64; _p = (_j % 64) // 4; _c = _j % 4
    _l = _lev_of_p[_p]
    _cx512[_j] = _c % 2; _cy512[_j] = _c // 2
    _Wl512[_j] = _W_of_lev[_l]; _Hl512[_j] = _H_of_lev[_l]
    _base512[_j] = _base_of_lev[_l]; _h512[_j] = _h

_SelX = np.zeros((256, 512), np.float32)
_SelY = np.zeros((256, 512), np.float32)
_SelA = np.zeros((128, 512), np.float32)
for _j in range(512):
    _h = _j // 64; _p = (_j % 64) // 4
    _SelX[_h * 32 + _p * 2 + 0, _j] = 1.0
    _SelY[_h * 32 + _p * 2 + 1, _j] = 1.0
    _SelA[_h * 16 + _p, _j] = 1.0
_Msum = np.kron(np.eye(8, dtype=np.float32), np.ones((16, 16), np.float32))

_C256 = np.stack([_dim256, _xybit256]).astype(np.float32)
_C512 = np.stack([_cx512, _cy512, _Wl512, _Hl512, _base512, _h512]).astype(np.float32)
_BOFF = ((np.arange(NQ) // LQ) * (LV * H)).astype(np.float32)[:, None]


def _proj_body(x_ref, w_ref, b_ref, o_ref):
    o_ref[...] = jnp.dot(x_ref[...], w_ref[...],
                         preferred_element_type=jnp.float32) + b_ref[...]


def _value_proj(value2d, Wv, bv):
    m = value2d.shape[0]
    blk = 1024
    grid = m // blk
    return pl.pallas_call(
        _proj_body,
        grid=(grid,),
        in_specs=[
            pl.BlockSpec((blk, E), lambda i: (i, 0)),
            pl.BlockSpec((E, E), lambda i: (0, 0)),
            pl.BlockSpec((1, E), lambda i: (0, 0)),
        ],
        out_specs=pl.BlockSpec((blk, E), lambda i: (i, 0)),
        out_shape=jax.ShapeDtypeStruct((m, E), jnp.float32),
    )(value2d, Wv, bv.reshape(1, E))


def _prep_body(q_ref, rp_ref, boff_ref, ws_ref, bso_ref, wa_ref, ba_ref,
               msum_ref, selx_ref, sely_ref, sela_ref, c256_ref, c512_ref,
               idx_ref, w_ref):
    q = q_ref[...]
    rp = rp_ref[...]
    so = jnp.dot(q, ws_ref[...], preferred_element_type=jnp.float32) + bso_ref[...]
    logits = jnp.dot(q, wa_ref[...], preferred_element_type=jnp.float32) + ba_ref[...]
    e = jnp.exp(logits)
    aw = e / jnp.dot(e, msum_ref[...], preferred_element_type=jnp.float32)

    dim = c256_ref[0:1, :]
    xybit = c256_ref[1:2, :]
    cxy = jnp.where(xybit == 1.0, rp[:, 1:2], rp[:, 0:1])
    cwh = jnp.where(xybit == 1.0, rp[:, 3:4], rp[:, 2:3])
    loc = cxy + ((so * 0.25) * cwh) * 0.5
    g = 2.0 * loc - 1.0
    pix = (g + 1.0) * (dim * 0.5) - 0.5
    f0 = jnp.floor(pix)
    fr = pix - f0

    selx = selx_ref[...]; sely = sely_ref[...]
    x0e = jnp.dot(f0, selx, preferred_element_type=jnp.float32)
    y0e = jnp.dot(f0, sely, preferred_element_type=jnp.float32)
    fxe = jnp.dot(fr, selx, preferred_element_type=jnp.float32)
    fye = jnp.dot(fr, sely, preferred_element_type=jnp.float32)

    cx = c512_ref[0:1, :]; cy = c512_ref[1:2, :]
    wl = c512_ref[2:3, :]; hl = c512_ref[3:4, :]
    basel = c512_ref[4:5, :]; hlane = c512_ref[5:6, :]

    xc = x0e + cx
    yc = y0e + cy
    valid = ((xc >= 0.0) & (xc <= wl - 1.0) & (yc >= 0.0) & (yc <= hl - 1.0))
    xcc = jnp.clip(xc, 0.0, wl - 1.0)
    ycc = jnp.clip(yc, 0.0, hl - 1.0)
    wx = jnp.where(cx == 1.0, fxe, 1.0 - fxe)
    wy = jnp.where(cy == 1.0, fye, 1.0 - fye)
    awe = jnp.dot(aw, sela_ref[...], preferred_element_type=jnp.float32)
    w_ref[...] = wx * wy * awe * jnp.where(valid, 1.0, 0.0)
    idxf = (basel + ycc * wl + xcc) * 8.0 + hlane + boff_ref[...]
    idx_ref[...] = idxf.astype(jnp.int32)


def _prep(q2, rp2, consts):
    blk = 720
    grid = NQ // blk
    ws, bso, wa, ba, msum, selx, sely, sela, c256, c512, boff = consts
    return pl.pallas_call(
        _prep_body,
        grid=(grid,),
        in_specs=[
            pl.BlockSpec((blk, E), lambda i: (i, 0)),
            pl.BlockSpec((blk, 4), lambda i: (i, 0)),
            pl.BlockSpec((blk, 1), lambda i: (i, 0)),
            pl.BlockSpec((E, 256), lambda i: (0, 0)),
            pl.BlockSpec((1, 256), lambda i: (0, 0)),
            pl.BlockSpec((E, 128), lambda i: (0, 0)),
            pl.BlockSpec((1, 128), lambda i: (0, 0)),
            pl.BlockSpec((128, 128), lambda i: (0, 0)),
            pl.BlockSpec((256, 512), lambda i: (0, 0)),
            pl.BlockSpec((256, 512), lambda i: (0, 0)),
            pl.BlockSpec((128, 512), lambda i: (0, 0)),
            pl.BlockSpec((2, 256), lambda i: (0, 0)),
            pl.BlockSpec((6, 512), lambda i: (0, 0)),
        ],
        out_specs=[
            pl.BlockSpec((blk, 512), lambda i: (i, 0)),
            pl.BlockSpec((blk, 512), lambda i: (i, 0)),
        ],
        out_shape=[
            jax.ShapeDtypeStruct((NQ, 512), jnp.int32),
            jax.ShapeDtypeStruct((NQ, 512), jnp.float32),
        ],
    )(q2, rp2, boff, ws, bso, wa, ba, msum, selx, sely, sela, c256, c512)


def _bcast_lane(vec, j):
    idx = jnp.full((16, 1), j, jnp.int32)
    dnums = lax.GatherDimensionNumbers(
        offset_dims=(), collapsed_slice_dims=(0,), start_index_map=(0,))
    return lax.gather(vec, idx, dnums, (1,),
                      mode=lax.GatherScatterMode.PROMISE_IN_BOUNDS)

def _sc_body(table_hbm, idx_hbm, w_hbm, out_hbm, idx_v, w_v, rows_v, out_v, sem):
    wid = lax.axis_index("s") * 2 + lax.axis_index("c")
    rbase = wid * RPW

    def group(gi, _):
        r0 = rbase + gi * GSZ
        k0 = r0 * KPR
        pltpu.sync_copy(idx_hbm.at[pl.ds(k0, KG)], idx_v)
        pltpu.sync_copy(w_hbm.at[pl.ds(k0, KG)], w_v)
        descs = []
        for j in range(NCH):
            descs.append(pltpu.async_copy(
                table_hbm.at[idx_v.at[pl.ds(j * 128, 128)]],
                rows_v.at[pl.ds(j * 128, 128)], sem))
        for d in descs:
            d.wait()

        def row(r, _):
            r64 = r * KPR
            acc0 = jnp.zeros((16,), jnp.float32)
            acc1 = jnp.zeros((16,), jnp.float32)
            for k16 in range(KPR // 16):
                wv = w_v[pl.ds(r64 + k16 * 16, 16)]
                for j in range(16):
                    kk = r64 + k16 * 16 + j
                    wb = _bcast_lane(wv, j)
                    acc0 = acc0 + wb * rows_v[kk, pl.ds(0, 16)]
                    acc1 = acc1 + wb * rows_v[kk, pl.ds(16, 16)]
            out_v[pl.ds(r * D, 16)] = acc0
            out_v[pl.ds(r * D + 16, 16)] = acc1
            return 0

        lax.fori_loop(0, GSZ, row, 0)
        pltpu.sync_copy(out_v, out_hbm.at[pl.ds(r0 * D, GSZ * D)])
        return 0

    lax.fori_loop(0, NGRP, group, 0)


def _sc_gather(table, idx2d, wflat):
    mesh = plsc.VectorSubcoreMesh(core_axis_name="c", subcore_axis_name="s")
    f = pl.kernel(
        _sc_body,
        out_type=jax.ShapeDtypeStruct((NR * D,), jnp.float32),
        mesh=mesh,
        compiler_params=pltpu.CompilerParams(use_tc_tiling_on_sc=False),
        scratch_types=[
            pltpu.VMEM((KG,), jnp.int32),
            pltpu.VMEM((KG,), jnp.float32),
            pltpu.VMEM((KG, D), jnp.float32),
            pltpu.VMEM((GSZ * D,), jnp.float32),
            pltpu.SemaphoreType.DMA,
        ],
    )
    return f(table, idx2d, wflat)


def _out_proj(core2d, Wo, bo):
    return pl.pallas_call(
        _proj_body,
        grid=(5,),
        in_specs=[
            pl.BlockSpec((720, E), lambda i: (i, 0)),
            pl.BlockSpec((E, E), lambda i: (0, 0)),
            pl.BlockSpec((1, E), lambda i: (0, 0)),
        ],
        out_specs=pl.BlockSpec((720, E), lambda i: (i, 0)),
        out_shape=jax.ShapeDtypeStruct((NQ, E), jnp.float32),
    )(core2d, Wo, bo.reshape(1, E))


def kernel(query, reference_points, value, value_spatial_shapes, Wv, bv, Ws, bso, Wa, ba, Wo, bo):
    del value_spatial_shapes
    q2 = query.reshape(NQ, E)
    rp2 = reference_points.reshape(NQ, 4)

    v2 = _value_proj(value.reshape(B * LV, E), Wv, bv)
    table = v2.reshape(TROWS, D)

    consts = (Ws, bso.reshape(1, 256), Wa, ba.reshape(1, 128),
              jnp.asarray(_Msum), jnp.asarray(_SelX), jnp.asarray(_SelY),
              jnp.asarray(_SelA), jnp.asarray(_C256), jnp.asarray(_C512),
              jnp.asarray(_BOFF))
    idx, w = _prep(q2, rp2, consts)

    core = _sc_gather(table, idx.reshape(NK), w.reshape(NK))
    out = _out_proj(core.reshape(NQ, E), Wo, bo)
    return out.reshape(B, LQ, E)
